# baseline (speedup 1.0000x reference)
import functools

import jax
import jax.numpy as jnp
from jax.experimental import pallas as pl

_NEG = float("-inf")


@functools.lru_cache(maxsize=None)
def _fixed_rand(shape):
    with jax.ensure_compile_time_eval():
        return jax.random.uniform(jax.random.key(42), shape,
                                  dtype=jnp.float32)


def _body(num_samples, adj_ref, rand_ref, x_all_ref, x_rows_ref,
          wst_ref, bs_ref, wnt_ref, bn_ref, out_ref):
    R, C = adj_ref.shape
    W = 640
    F = -(-C // W)
    neg = jnp.float32(_NEG)
    m1 = jnp.full((R, W), _NEG, jnp.float32)
    m2 = jnp.full((R, W), _NEG, jnp.float32)
    m3 = jnp.full((R, W), _NEG, jnp.float32)
    for f in range(F):
        lo = f * W
        hi = min(lo + W, C)
        c = jnp.where(adj_ref[:, lo:hi] > 0, rand_ref[:, lo:hi], neg)
        if hi - lo < W:
            c = jnp.concatenate(
                [c, jnp.full((R, W - (hi - lo)), _NEG, jnp.float32)], axis=1)
        b1 = c > m1
        b2 = c > m2
        b3 = c > m3
        m3 = jnp.where(b2, m2, jnp.where(b3, c, m3))
        m2 = jnp.where(b1, m1, jnp.where(b2, c, m2))
        m1 = jnp.where(b1, c, m1)
    cand = jnp.concatenate([m1, m2, m3], axis=1)
    col = jax.lax.broadcasted_iota(jnp.int32, (R, 3 * W), 1)
    big = jnp.int32(2**30)
    for _ in range(num_samples - 1):
        m = jnp.max(cand, axis=1, keepdims=True)
        pos = jnp.min(jnp.where(cand == m, col, big), axis=1, keepdims=True)
        cand = jnp.where((col == pos) & (m > neg), neg, cand)
    v10 = jnp.max(cand, axis=1, keepdims=True)
    score = jnp.where(adj_ref[...] > 0, rand_ref[...], neg)
    sel = ((score >= v10) & (score != neg)).astype(jnp.float32)
    cnt = jnp.sum(sel, axis=1, keepdims=True)
    inv = 1.0 / jnp.maximum(cnt, 1.0)
    agg = jnp.dot(sel, x_all_ref[...],
                  preferred_element_type=jnp.float32) * inv
    out_ref[...] = (
        jnp.dot(x_rows_ref[...], wst_ref[...],
                preferred_element_type=jnp.float32) + bs_ref[...]
        + jnp.dot(agg, wnt_ref[...],
                  preferred_element_type=jnp.float32) + bn_ref[...])


def kernel(x, adj, W_self, b_self, W_neigh, b_neigh):
    N, D = x.shape
    num_samples = 10
    rand = _fixed_rand(tuple(adj.shape))
    for R in (80, 64, 40, 16, 8):
        if N % R == 0:
            break
    grid = (N // R,)
    body = functools.partial(_body, num_samples)
    return pl.pallas_call(
        body,
        grid=grid,
        in_specs=[
            pl.BlockSpec((R, N), lambda i: (i, 0)),
            pl.BlockSpec((R, N), lambda i: (i, 0)),
            pl.BlockSpec((N, D), lambda i: (0, 0)),
            pl.BlockSpec((R, D), lambda i: (i, 0)),
            pl.BlockSpec((D, D), lambda i: (0, 0)),
            pl.BlockSpec((1, D), lambda i: (0, 0)),
            pl.BlockSpec((D, D), lambda i: (0, 0)),
            pl.BlockSpec((1, D), lambda i: (0, 0)),
        ],
        out_specs=pl.BlockSpec((R, D), lambda i: (i, 0)),
        out_shape=jax.ShapeDtypeStruct((N, D), jnp.float32),
    )(adj, rand, x, x, W_self.T, b_self[None, :], W_neigh.T, b_neigh[None, :])


# speedup vs baseline: 1.6678x; 1.6678x over previous
import functools

import jax
import jax.numpy as jnp
from jax.experimental import pallas as pl

_NEG = float("-inf")


@functools.lru_cache(maxsize=None)
def _fixed_rand(shape):
    with jax.ensure_compile_time_eval():
        return jax.random.uniform(jax.random.key(42), shape,
                                  dtype=jnp.float32)


def _body(num_samples, adj_ref, rand_ref, x_all_ref, x_rows_ref,
          wst_ref, bs_ref, wnt_ref, bn_ref, out_ref):
    R, C = adj_ref.shape
    W = 640
    F = -(-C // W)
    neg = jnp.float32(_NEG)
    m1 = jnp.full((R, W), _NEG, jnp.float32)
    m2 = jnp.full((R, W), _NEG, jnp.float32)
    for f in range(F):
        lo = f * W
        hi = min(lo + W, C)
        c = jnp.where(adj_ref[:, lo:hi] > 0, rand_ref[:, lo:hi], neg)
        if hi - lo < W:
            c = jnp.concatenate(
                [c, jnp.full((R, W - (hi - lo)), _NEG, jnp.float32)], axis=1)
        b1 = c > m1
        b2 = c > m2
        m2 = jnp.where(b1, m1, jnp.where(b2, c, m2))
        m1 = jnp.where(b1, c, m1)
    cand = jnp.concatenate([m1, m2], axis=1)
    for _ in range(num_samples - 1):
        m = jnp.max(cand, axis=1, keepdims=True)
        cand = jnp.where(cand == m, neg, cand)
    v10 = jnp.max(cand, axis=1, keepdims=True)
    tv = jnp.maximum(v10, 0.0)
    sel = ((rand_ref[...] >= tv) & (adj_ref[...] > 0)).astype(jnp.float32)
    cnt = jnp.sum(sel, axis=1, keepdims=True)
    inv = 1.0 / jnp.maximum(cnt, 1.0)
    agg = jnp.dot(sel, x_all_ref[...],
                  preferred_element_type=jnp.float32) * inv
    out_ref[...] = (
        jnp.dot(x_rows_ref[...], wst_ref[...],
                preferred_element_type=jnp.float32) + bs_ref[...]
        + jnp.dot(agg, wnt_ref[...],
                  preferred_element_type=jnp.float32) + bn_ref[...])


def kernel(x, adj, W_self, b_self, W_neigh, b_neigh):
    N, D = x.shape
    num_samples = 10
    rand = _fixed_rand(tuple(adj.shape))
    for R in (80, 64, 40, 16, 8):
        if N % R == 0:
            break
    grid = (N // R,)
    body = functools.partial(_body, num_samples)
    return pl.pallas_call(
        body,
        grid=grid,
        in_specs=[
            pl.BlockSpec((R, N), lambda i: (i, 0)),
            pl.BlockSpec((R, N), lambda i: (i, 0)),
            pl.BlockSpec((N, D), lambda i: (0, 0)),
            pl.BlockSpec((R, D), lambda i: (i, 0)),
            pl.BlockSpec((D, D), lambda i: (0, 0)),
            pl.BlockSpec((1, D), lambda i: (0, 0)),
            pl.BlockSpec((D, D), lambda i: (0, 0)),
            pl.BlockSpec((1, D), lambda i: (0, 0)),
        ],
        out_specs=pl.BlockSpec((R, D), lambda i: (i, 0)),
        out_shape=jax.ShapeDtypeStruct((N, D), jnp.float32),
    )(adj, rand, x, x, W_self.T, b_self[None, :], W_neigh.T, b_neigh[None, :])
